# baseline (speedup 1.0000x reference)
import functools
import math

import jax
import jax.numpy as jnp
from jax import lax
from jax.experimental import pallas as pl
from jax.experimental.pallas import tpu as pltpu
from jax.experimental.pallas import tpu_sc as plsc

ROWS = 16
COLS = 2048
L = 16
NVEC = COLS // L
INV_N = 1.0 / (ROWS * COLS)

_mesh = plsc.VectorSubcoreMesh(core_axis_name="c", subcore_axis_name="s")


def _lane_allreduce(vec, buf):
    for off in (1, 2, 4, 8):
        buf[pl.ds(0, L)] = vec
        buf[pl.ds(L, L)] = vec
        vec = vec + buf[pl.ds(off, L)]
    return vec


@functools.partial(
    pl.kernel,
    mesh=_mesh,
    out_type=jax.ShapeDtypeStruct((L,), jnp.float32),
    scratch_types=[
        pltpu.VMEM((COLS,), jnp.float32),
        pltpu.VMEM((COLS,), jnp.float32),
        pltpu.VMEM((2 * L,), jnp.float32),
        pltpu.VMEM((L,), jnp.float32),
        pltpu.VMEM((ROWS * L,), jnp.float32),
        pltpu.VMEM_SHARED((ROWS * L,), jnp.float32),
    ],
)
def _corr_mse_kernel(pred_hbm, label_hbm, out_hbm,
                     pred_v, label_v, rot_v, part_v, fin_v, shared):
    c = lax.axis_index("c")
    s = lax.axis_index("s")

    @pl.when(c == 0)
    def _compute_row():
        pltpu.sync_copy(pred_hbm.at[s], pred_v)
        pltpu.sync_copy(label_hbm.at[s], label_v)

        zero = jnp.zeros((L,), jnp.float32)

        def body(j, carry):
            sse, spos, sneg, nones = carry
            p = pred_v[pl.ds(j * L, L)]
            lab = label_v[pl.ds(j * L, L)]
            d = p - lab
            e = jnp.exp(p)
            em = jnp.exp(-p)
            sse = sse + d * d
            spos = spos + lab * em
            sneg = sneg + (1.0 - lab) * e
            nones = nones + lab
            return sse, spos, sneg, nones

        sse, spos, sneg, nones = lax.fori_loop(
            0, NVEC, body, (zero, zero, zero, zero))

        sse_t = _lane_allreduce(sse, rot_v)
        s_pos = _lane_allreduce(spos, rot_v)
        s_neg = _lane_allreduce(sneg, rot_v)
        n_one = _lane_allreduce(nones, rot_v)
        n_zero = float(COLS) - n_one

        loss_both = s_pos * s_neg / jnp.maximum(n_one * n_zero, 1.0)
        loss_all_zero = s_neg * math.exp(-1.0) / jnp.maximum(n_zero, 1.0)
        loss_all_one = s_pos / jnp.maximum(n_one, 1.0)
        row_loss = jnp.where(
            n_one == 0.0, loss_all_zero,
            jnp.where(n_zero == 0.0, loss_all_one, loss_both))

        part_v[...] = row_loss + sse_t * INV_N
        pltpu.sync_copy(part_v, shared.at[pl.ds(s * L, L)])

    plsc.subcore_barrier()

    @pl.when((c == 0) & (s == 0))
    def _combine():
        pltpu.sync_copy(shared, fin_v)

        def body(i, acc):
            return acc + fin_v[pl.ds(i * L, L)]

        acc = lax.fori_loop(0, ROWS, body, jnp.zeros((L,), jnp.float32))
        part_v[...] = acc
        pltpu.sync_copy(part_v, out_hbm)


def kernel(pred, label):
    out = _corr_mse_kernel(pred, label)
    return out[0]


# speedup vs baseline: 1.1223x; 1.1223x over previous
import functools
import math

import jax
import jax.numpy as jnp
from jax import lax
from jax.experimental import pallas as pl
from jax.experimental.pallas import tpu as pltpu
from jax.experimental.pallas import tpu_sc as plsc

ROWS = 16
COLS = 2048
L = 16
UNROLL = 4
NCHUNK = COLS // (L * UNROLL)
INV_N = 1.0 / (ROWS * COLS)

_mesh = plsc.VectorSubcoreMesh(core_axis_name="c", subcore_axis_name="s",
                               num_cores=1)


def _lane_allreduce(vec):
    lane = lax.iota(jnp.int32, L)
    for off in (1, 2, 4, 8):
        vec = vec + vec.at[lane ^ off].get(
            mode=lax.GatherScatterMode.PROMISE_IN_BOUNDS)
    return vec


@functools.partial(
    pl.kernel,
    mesh=_mesh,
    out_type=jax.ShapeDtypeStruct((L,), jnp.float32),
    scratch_types=[
        pltpu.VMEM((COLS,), jnp.float32),
        pltpu.VMEM((COLS,), jnp.float32),
        pltpu.VMEM((L,), jnp.float32),
        pltpu.VMEM((ROWS * L,), jnp.float32),
        pltpu.VMEM_SHARED((ROWS * L,), jnp.float32),
        pltpu.SemaphoreType.DMA,
        pltpu.SemaphoreType.DMA,
    ],
)
def _corr_mse_kernel(pred_hbm, label_hbm, out_hbm,
                     pred_v, label_v, part_v, fin_v, shared,
                     sem_p, sem_l):
    c = lax.axis_index("c")
    s = lax.axis_index("s")

    @pl.when(c == 0)
    def _compute_row():
        cp_p = pltpu.async_copy(pred_hbm.at[s], pred_v, sem_p)
        cp_l = pltpu.async_copy(label_hbm.at[s], label_v, sem_l)
        cp_p.wait()
        cp_l.wait()

        zero = jnp.zeros((L,), jnp.float32)

        def body(j, carry):
            sse, spos, sneg, nones = carry
            base = j * (L * UNROLL)
            for u in range(UNROLL):
                p = pred_v[pl.ds(base + u * L, L)]
                lab = label_v[pl.ds(base + u * L, L)]
                nlab = 1.0 - lab
                d = p - lab
                t = jnp.exp(p * (nlab - lab))
                sse = sse + d * d
                spos = spos + lab * t
                sneg = sneg + nlab * t
                nones = nones + lab
            return sse, spos, sneg, nones

        sse, spos, sneg, nones = lax.fori_loop(
            0, NCHUNK, body, (zero, zero, zero, zero))

        sse_t = _lane_allreduce(sse)
        s_pos = _lane_allreduce(spos)
        s_neg = _lane_allreduce(sneg)
        n_one = _lane_allreduce(nones)
        n_zero = float(COLS) - n_one

        loss_both = s_pos * s_neg / jnp.maximum(n_one * n_zero, 1.0)
        loss_all_zero = s_neg * math.exp(-1.0) / jnp.maximum(n_zero, 1.0)
        loss_all_one = s_pos / jnp.maximum(n_one, 1.0)
        row_loss = jnp.where(
            n_one == 0.0, loss_all_zero,
            jnp.where(n_zero == 0.0, loss_all_one, loss_both))

        part_v[...] = row_loss + sse_t * INV_N
        pltpu.sync_copy(part_v, shared.at[pl.ds(s * L, L)])

    plsc.subcore_barrier()

    @pl.when((c == 0) & (s == 0))
    def _combine():
        pltpu.sync_copy(shared, fin_v)
        acc = fin_v[pl.ds(0, L)]
        for i in range(1, ROWS):
            acc = acc + fin_v[pl.ds(i * L, L)]
        part_v[...] = acc
        pltpu.sync_copy(part_v, out_hbm)


def kernel(pred, label):
    out = _corr_mse_kernel(pred, label)
    return out[0]
